# baseline (speedup 1.0000x reference)
import functools

import jax
import jax.numpy as jnp
from jax import lax
from jax.experimental import pallas as pl
from jax.experimental.pallas import tpu as pltpu
from jax.experimental.pallas import tpu_sc as plsc

NC = 2
NS = 16
L = 16
NW = NC * NS

B = 16384
EMB = 64
BW = B // NW
CHUNK = 128
NCHUNK = BW // CHUNK


def _mf_body(u_id_hbm, i_id_hbm, user_emb_hbm, user_bias_hbm, item_emb_hbm,
             item_bias_hbm, mean_hbm, out_hbm,
             uidx_v, iidx_v, uhi_v, ihi_v, ulo_v, ilo_v,
             urows_v, irows_v, ubias_v, ibias_v, out_v, mean_v, sem):
    wid = lax.axis_index("s") * NC + lax.axis_index("c")
    base = wid * BW

    for j in range(NCHUNK):
        pltpu.sync_copy(u_id_hbm.at[pl.ds(base + j * CHUNK, CHUNK)],
                        uidx_v.at[j])
        pltpu.sync_copy(i_id_hbm.at[pl.ds(base + j * CHUNK, CHUNK)],
                        iidx_v.at[j])
    pltpu.sync_copy(mean_hbm, mean_v.at[pl.ds(0, 1)])

    mask15 = jnp.full((L,), 15, jnp.int32)
    for j in range(NCHUNK):
        for k in range(CHUNK // L):
            sl = pl.ds(k * L, L)
            fl = pl.ds(j * CHUNK + k * L, L)
            uv = uidx_v[j, sl]
            iv = iidx_v[j, sl]
            uhi_v[j, sl] = lax.shift_right_logical(uv, 4)
            ihi_v[j, sl] = lax.shift_right_logical(iv, 4)
            ulo_v[fl] = lax.bitwise_and(uv, mask15)
            ilo_v[fl] = lax.bitwise_and(iv, mask15)

    copies = []
    for j in range(NCHUNK):
        sl = pl.ds(j * CHUNK, CHUNK)
        copies.append(pltpu.async_copy(user_emb_hbm.at[uidx_v.at[j]],
                                       urows_v.at[sl], sem))
        copies.append(pltpu.async_copy(item_emb_hbm.at[iidx_v.at[j]],
                                       irows_v.at[sl], sem))
        copies.append(pltpu.async_copy(user_bias_hbm.at[uhi_v.at[j]],
                                       ubias_v.at[sl], sem))
        copies.append(pltpu.async_copy(item_bias_hbm.at[ihi_v.at[j]],
                                       ibias_v.at[sl], sem))
    for c in copies:
        c.wait()

    iota16 = lax.iota(jnp.int32, L)
    mean_s = mean_v[pl.ds(0, L)][0]

    def group_body(g, carry):
        rows = g * L + iota16
        acc = jnp.zeros((L,), jnp.float32)

        def d_body(d, acc):
            dcol = jnp.full((L,), d, jnp.int32)
            u = plsc.load_gather(urows_v, [rows, dcol])
            iv = plsc.load_gather(irows_v, [rows, dcol])
            return acc + u * iv

        acc = lax.fori_loop(0, EMB, d_body, acc)
        sl = pl.ds(g * L, L)
        ub = plsc.load_gather(ubias_v, [rows, ulo_v[sl]])
        ib = plsc.load_gather(ibias_v, [rows, ilo_v[sl]])
        out_v[sl] = acc + ub + ib + mean_s
        return carry

    lax.fori_loop(0, BW // L, group_body, 0)

    pltpu.sync_copy(out_v, out_hbm.at[pl.ds(base, BW)])


@jax.jit
def _mf(u_id, i_id, user_emb, user_bias, item_emb, item_bias, mean):
    nu = user_bias.shape[0]
    ni = item_bias.shape[0]
    ub16 = user_bias.reshape(nu // L, L)
    ib16 = item_bias.reshape(ni // L, L)
    return pl.kernel(
        _mf_body,
        out_type=jax.ShapeDtypeStruct((B,), jnp.float32),
        mesh=plsc.VectorSubcoreMesh(core_axis_name="c", subcore_axis_name="s",
                                    num_cores=NC, num_subcores=NS),
        scratch_types=[
            pltpu.VMEM((NCHUNK, CHUNK), jnp.int32),
            pltpu.VMEM((NCHUNK, CHUNK), jnp.int32),
            pltpu.VMEM((NCHUNK, CHUNK), jnp.int32),
            pltpu.VMEM((NCHUNK, CHUNK), jnp.int32),
            pltpu.VMEM((BW,), jnp.int32),
            pltpu.VMEM((BW,), jnp.int32),
            pltpu.VMEM((BW, EMB), jnp.float32),
            pltpu.VMEM((BW, EMB), jnp.float32),
            pltpu.VMEM((BW, L), jnp.float32),
            pltpu.VMEM((BW, L), jnp.float32),
            pltpu.VMEM((BW,), jnp.float32),
            pltpu.VMEM((L,), jnp.float32),
            pltpu.SemaphoreType.DMA,
        ],
        compiler_params=pltpu.CompilerParams(needs_layout_passes=False,
                                             use_tc_tiling_on_sc=False),
    )(u_id, i_id, user_emb, ub16, item_emb, ib16, mean)


def kernel(u_id, i_id, user_emb, user_bias, item_emb, item_bias, mean):
    return _mf(u_id, i_id, user_emb, user_bias, item_emb, item_bias, mean)
